# Initial kernel scaffold; baseline (speedup 1.0000x reference)
import jax
import jax.numpy as jnp
from jax.experimental import pallas as pl


def kernel(x, W_emb, W_lin, bias, W0, b0, W1, b1, W2, b2):
    raise NotImplementedError("write your pallas kernel here")


# speedup vs baseline: 1.4352x; 1.4352x over previous
import functools

import jax
import jax.numpy as jnp
import numpy as np
from jax import lax
from jax.experimental import pallas as pl
from jax.experimental.pallas import tpu as pltpu
from jax.experimental.pallas import tpu_sc as plsc

FIELD_DIMS = [100000] * 26
NUM_FIELDS = len(FIELD_DIMS)
TOTAL_DIM = sum(FIELD_DIMS)
EMBED_DIM = 128
BATCH = 16384
EMBED_OUT = NUM_FIELDS * EMBED_DIM
OFFSETS = np.concatenate(([0], np.cumsum(FIELD_DIMS)[:-1])).astype(np.int32)
BN_SCALE = float(1.0 / np.sqrt(1.0 + 1e-5))

NC, NS = 2, 16
NW = NC * NS
TOTAL_ROWS = BATCH * NUM_FIELDS
ROWS_PER_W = TOTAL_ROWS // NW
CSZ = 128
CHUNKS = ROWS_PER_W // CSZ


def _sc_gather(w_emb, w_lin_flat, idx3):
    mesh = plsc.VectorSubcoreMesh(core_axis_name="c", subcore_axis_name="s")

    @functools.partial(
        pl.kernel,
        out_type=(
            jax.ShapeDtypeStruct((TOTAL_ROWS, EMBED_DIM), jnp.float32),
            jax.ShapeDtypeStruct((TOTAL_ROWS,), jnp.float32),
        ),
        mesh=mesh,
        scratch_types=(
            pltpu.VMEM((CHUNKS, CSZ), jnp.int32),
            pltpu.VMEM((CSZ, EMBED_DIM), jnp.float32),
            pltpu.VMEM((CSZ,), jnp.float32),
            pltpu.SemaphoreType.DMA,
            pltpu.SemaphoreType.DMA,
        ),
    )
    def k(table_hbm, linw_hbm, idx_hbm, emb_out, lin_out,
          idx_v, rows_v, linv_v, sem_e, sem_l):
        wid = lax.axis_index("s") * NC + lax.axis_index("c")
        base = wid * ROWS_PER_W
        pltpu.sync_copy(idx_hbm.at[wid], idx_v)

        def body(c, carry):
            irow = idx_v.at[c]
            ge = pltpu.async_copy(table_hbm.at[irow], rows_v, sem_e)
            gl = pltpu.async_copy(linw_hbm.at[irow], linv_v, sem_l)
            ge.wait()
            gl.wait()
            pltpu.sync_copy(rows_v, emb_out.at[pl.ds(base + c * CSZ, CSZ)])
            pltpu.sync_copy(linv_v, lin_out.at[pl.ds(base + c * CSZ, CSZ)])
            return carry

        lax.fori_loop(0, CHUNKS, body, 0)

    return k(w_emb, w_lin_flat, idx3)


def _tc_body(emb_ref, linv_ref, w0_ref, b0_ref, w1_ref, b1_ref, w2_ref,
             c0_ref, out_ref):
    h = emb_ref[...]
    s = h[:, 0:EMBED_DIM]
    ssq = s * s
    for f in range(1, NUM_FIELDS):
        v = h[:, f * EMBED_DIM:(f + 1) * EMBED_DIM]
        s = s + v
        ssq = ssq + v * v
    fm = 0.5 * jnp.sum(s * s - ssq, axis=1, keepdims=True)
    lin = jnp.sum(linv_ref[...], axis=1, keepdims=True)
    y0 = jnp.dot(h.astype(jnp.bfloat16), w0_ref[...],
                 preferred_element_type=jnp.float32)
    y0 = jnp.maximum((y0 + b0_ref[...]) * BN_SCALE, 0.0)
    y1 = jnp.dot(y0.astype(jnp.bfloat16), w1_ref[...],
                 preferred_element_type=jnp.float32)
    y1 = jnp.maximum((y1 + b1_ref[...]) * BN_SCALE, 0.0)
    y2 = jnp.sum(y1 * w2_ref[...], axis=1, keepdims=True)
    logit = lin + fm + y2 + c0_ref[...]
    out_ref[...] = jax.nn.sigmoid(logit)


def _tc_fused(emb, linv, w0b, b0r, w1b, b1r, w2r, c0, block_b, interpret=False):
    grid = (BATCH // block_b,)
    return pl.pallas_call(
        _tc_body,
        grid=grid,
        in_specs=[
            pl.BlockSpec((block_b, EMBED_OUT), lambda i: (i, 0)),
            pl.BlockSpec((block_b, NUM_FIELDS), lambda i: (i, 0)),
            pl.BlockSpec((EMBED_OUT, 1024), lambda i: (0, 0)),
            pl.BlockSpec((1, 1024), lambda i: (0, 0)),
            pl.BlockSpec((1024, 512), lambda i: (0, 0)),
            pl.BlockSpec((1, 512), lambda i: (0, 0)),
            pl.BlockSpec((1, 512), lambda i: (0, 0)),
            pl.BlockSpec((1, 1), lambda i: (0, 0)),
        ],
        out_specs=pl.BlockSpec((block_b, 1), lambda i: (i, 0)),
        out_shape=jax.ShapeDtypeStruct((BATCH, 1), jnp.float32),
        interpret=interpret,
    )(emb, linv, w0b, b0r, w1b, b1r, w2r, c0)


def kernel(x, W_emb, W_lin, bias, W0, b0, W1, b1, W2, b2):
    xo = x + jnp.asarray(OFFSETS)[None, :]
    idx3 = xo.reshape(NW, CHUNKS, CSZ)
    emb_flat, lin_flat = _sc_gather(W_emb, W_lin.reshape(-1), idx3)
    emb = emb_flat.reshape(BATCH, EMBED_OUT)
    linv = lin_flat.reshape(BATCH, NUM_FIELDS)
    out = _tc_fused(
        emb, linv,
        W0.astype(jnp.bfloat16), b0.reshape(1, -1),
        W1.astype(jnp.bfloat16), b1.reshape(1, -1),
        W2.reshape(1, -1), (bias + b2).reshape(1, 1),
        block_b=512,
    )
    return out.reshape(BATCH)


# speedup vs baseline: 1.4758x; 1.0283x over previous
import functools

import jax
import jax.numpy as jnp
import numpy as np
from jax import lax
from jax.experimental import pallas as pl
from jax.experimental.pallas import tpu as pltpu
from jax.experimental.pallas import tpu_sc as plsc

FIELD_DIMS = [100000] * 26
NUM_FIELDS = len(FIELD_DIMS)
TOTAL_DIM = sum(FIELD_DIMS)
EMBED_DIM = 128
BATCH = 16384
EMBED_OUT = NUM_FIELDS * EMBED_DIM
OFFSETS = np.concatenate(([0], np.cumsum(FIELD_DIMS)[:-1])).astype(np.int32)
BN_SCALE = float(1.0 / np.sqrt(1.0 + 1e-5))

NC, NS = 2, 16
NW = NC * NS
TOTAL_ROWS = BATCH * NUM_FIELDS
ROWS_PER_W = TOTAL_ROWS // NW
CSZ = 128
CHUNKS = ROWS_PER_W // CSZ


def _sc_gather(w_emb, w_lin_flat, idx3):
    mesh = plsc.VectorSubcoreMesh(core_axis_name="c", subcore_axis_name="s")

    @functools.partial(
        pl.kernel,
        out_type=(
            jax.ShapeDtypeStruct((TOTAL_ROWS, EMBED_DIM), jnp.float32),
            jax.ShapeDtypeStruct((TOTAL_ROWS,), jnp.float32),
        ),
        mesh=mesh,
        scratch_types=(
            pltpu.VMEM((CHUNKS, CSZ), jnp.int32),
            pltpu.VMEM((CSZ, EMBED_DIM), jnp.float32),
            pltpu.VMEM((CSZ, EMBED_DIM), jnp.float32),
            pltpu.VMEM((CSZ,), jnp.float32),
            pltpu.VMEM((CSZ,), jnp.float32),
            pltpu.SemaphoreType.DMA,
            pltpu.SemaphoreType.DMA,
            pltpu.SemaphoreType.DMA,
            pltpu.SemaphoreType.DMA,
        ),
    )
    def k(table_hbm, linw_hbm, idx_hbm, emb_out, lin_out,
          idx_v, rows0, rows1, lin0, lin1, sem_e0, sem_e1, sem_l0, sem_l1):
        wid = lax.axis_index("s") * NC + lax.axis_index("c")
        base = wid * ROWS_PER_W
        pltpu.sync_copy(idx_hbm.at[wid], idx_v)

        pltpu.async_copy(table_hbm.at[idx_v.at[0]], rows0, sem_e0)
        pltpu.async_copy(linw_hbm.at[idx_v.at[0]], lin0, sem_l0)

        def pair(i, carry):
            c = 2 * i
            i0 = idx_v.at[c]
            i1 = idx_v.at[c + 1]
            pltpu.async_copy(table_hbm.at[i1], rows1, sem_e1)
            pltpu.async_copy(linw_hbm.at[i1], lin1, sem_l1)
            pltpu.make_async_copy(table_hbm.at[i0], rows0, sem_e0).wait()
            pltpu.make_async_copy(linw_hbm.at[i0], lin0, sem_l0).wait()
            pltpu.sync_copy(rows0, emb_out.at[pl.ds(base + c * CSZ, CSZ)])
            pltpu.sync_copy(lin0, lin_out.at[pl.ds(base + c * CSZ, CSZ)])

            @pl.when(i + 1 < CHUNKS // 2)
            def _():
                i2 = idx_v.at[c + 2]
                pltpu.async_copy(table_hbm.at[i2], rows0, sem_e0)
                pltpu.async_copy(linw_hbm.at[i2], lin0, sem_l0)

            pltpu.make_async_copy(table_hbm.at[i1], rows1, sem_e1).wait()
            pltpu.make_async_copy(linw_hbm.at[i1], lin1, sem_l1).wait()
            pltpu.sync_copy(rows1, emb_out.at[pl.ds(base + (c + 1) * CSZ, CSZ)])
            pltpu.sync_copy(lin1, lin_out.at[pl.ds(base + (c + 1) * CSZ, CSZ)])
            return carry

        lax.fori_loop(0, CHUNKS // 2, pair, 0)

    return k(w_emb, w_lin_flat, idx3)


def _tc_body(emb_ref, linv_ref, w0_ref, b0_ref, w1_ref, b1_ref, w2_ref,
             c0_ref, smat_ref, ones_ref, out_ref):
    h = emb_ref[...]
    hb = h.astype(jnp.bfloat16)
    s = jnp.dot(hb, smat_ref[...], preferred_element_type=jnp.float32)
    ssqsum = jnp.dot(hb * hb, ones_ref[...],
                     preferred_element_type=jnp.float32)[:, 0:1]
    fm = 0.5 * (jnp.sum(s * s, axis=1, keepdims=True) - ssqsum)
    lin = jnp.sum(linv_ref[...], axis=1, keepdims=True)
    y0 = jnp.dot(hb, w0_ref[...], preferred_element_type=jnp.float32)
    y0 = jnp.maximum((y0 + b0_ref[...]) * BN_SCALE, 0.0)
    y1 = jnp.dot(y0.astype(jnp.bfloat16), w1_ref[...],
                 preferred_element_type=jnp.float32)
    y1 = jnp.maximum((y1 + b1_ref[...]) * BN_SCALE, 0.0)
    y2 = jnp.sum(y1 * w2_ref[...], axis=1, keepdims=True)
    logit = lin + fm + y2 + c0_ref[...]
    out_ref[...] = jax.nn.sigmoid(logit)


def _tc_fused(emb, linv, w0b, b0r, w1b, b1r, w2r, c0, block_b, interpret=False):
    grid = (BATCH // block_b,)
    smat = jnp.asarray(
        np.tile(np.eye(EMBED_DIM, dtype=np.float32), (NUM_FIELDS, 1)),
        dtype=jnp.bfloat16)
    ones8 = jnp.ones((EMBED_OUT, 8), dtype=jnp.bfloat16)
    return pl.pallas_call(
        _tc_body,
        grid=grid,
        in_specs=[
            pl.BlockSpec((block_b, EMBED_OUT), lambda i: (i, 0)),
            pl.BlockSpec((block_b, NUM_FIELDS), lambda i: (i, 0)),
            pl.BlockSpec((EMBED_OUT, 1024), lambda i: (0, 0)),
            pl.BlockSpec((1, 1024), lambda i: (0, 0)),
            pl.BlockSpec((1024, 512), lambda i: (0, 0)),
            pl.BlockSpec((1, 512), lambda i: (0, 0)),
            pl.BlockSpec((1, 512), lambda i: (0, 0)),
            pl.BlockSpec((1, 1), lambda i: (0, 0)),
            pl.BlockSpec((EMBED_OUT, EMBED_DIM), lambda i: (0, 0)),
            pl.BlockSpec((EMBED_OUT, 8), lambda i: (0, 0)),
        ],
        out_specs=pl.BlockSpec((block_b, 1), lambda i: (i, 0)),
        out_shape=jax.ShapeDtypeStruct((BATCH, 1), jnp.float32),
        interpret=interpret,
    )(emb, linv, w0b, b0r, w1b, b1r, w2r, c0, smat, ones8)


def kernel(x, W_emb, W_lin, bias, W0, b0, W1, b1, W2, b2):
    xo = x + jnp.asarray(OFFSETS)[None, :]
    idx3 = xo.reshape(NW, CHUNKS, CSZ)
    emb_flat, lin_flat = _sc_gather(W_emb, W_lin.reshape(-1), idx3)
    emb = emb_flat.reshape(BATCH, EMBED_OUT)
    linv = lin_flat.reshape(BATCH, NUM_FIELDS)
    out = _tc_fused(
        emb, linv,
        W0.astype(jnp.bfloat16), b0.reshape(1, -1),
        W1.astype(jnp.bfloat16), b1.reshape(1, -1),
        W2.reshape(1, -1), (bias + b2).reshape(1, 1),
        block_b=512,
    )
    return out.reshape(BATCH)


# speedup vs baseline: 2.1128x; 1.4317x over previous
import functools

import jax
import jax.numpy as jnp
import numpy as np
from jax import lax
from jax.experimental import pallas as pl
from jax.experimental.pallas import tpu as pltpu
from jax.experimental.pallas import tpu_sc as plsc

FIELD_DIMS = [100000] * 26
NUM_FIELDS = len(FIELD_DIMS)
TOTAL_DIM = sum(FIELD_DIMS)
EMBED_DIM = 128
BATCH = 16384
EMBED_OUT = NUM_FIELDS * EMBED_DIM
OFFSETS = np.concatenate(([0], np.cumsum(FIELD_DIMS)[:-1])).astype(np.int32)
BN_SCALE = float(1.0 / np.sqrt(1.0 + 1e-5))

NC, NS = 2, 16
NW = NC * NS
B_PER_W = BATCH // NW
CSZ = 128
KSUB = B_PER_W // CSZ
CHUNKS = KSUB * NUM_FIELDS
H0, H1 = 1024, 512
NAUG = H0 + EMBED_DIM


def _sc_gather(w_emb, w_lin_flat, idx3):
    mesh = plsc.VectorSubcoreMesh(core_axis_name="c", subcore_axis_name="s")

    @functools.partial(
        pl.kernel,
        out_type=(
            jax.ShapeDtypeStruct((NUM_FIELDS, BATCH, EMBED_DIM), jnp.float32),
            jax.ShapeDtypeStruct((BATCH,), jnp.float32),
        ),
        mesh=mesh,
        scratch_types=(
            pltpu.VMEM((CHUNKS, CSZ), jnp.int32),
            pltpu.VMEM((CSZ, EMBED_DIM), jnp.float32),
            pltpu.VMEM((CSZ, EMBED_DIM), jnp.float32),
            pltpu.VMEM((CSZ,), jnp.float32),
            pltpu.VMEM((CSZ,), jnp.float32),
            pltpu.VMEM((B_PER_W,), jnp.float32),
            pltpu.SemaphoreType.DMA,
            pltpu.SemaphoreType.DMA,
            pltpu.SemaphoreType.DMA,
            pltpu.SemaphoreType.DMA,
        ),
    )
    def k(table_hbm, linw_hbm, idx_hbm, emb_out, lin_out,
          idx_v, rows0, rows1, lin0, lin1, acc_v,
          sem_e0, sem_e1, sem_l0, sem_l1):
        wid = lax.axis_index("s") * NC + lax.axis_index("c")
        bbase = wid * B_PER_W
        pltpu.sync_copy(idx_hbm.at[wid], idx_v)
        for j in range(B_PER_W // 16):
            acc_v[pl.ds(j * 16, 16)] = jnp.zeros((16,), jnp.float32)

        def accum(linbuf, koff):
            for j in range(CSZ // 16):
                sl = pl.ds(koff + j * 16, 16)
                acc_v[sl] = acc_v[sl] + linbuf[pl.ds(j * 16, 16)]

        pltpu.async_copy(table_hbm.at[idx_v.at[0]], rows0, sem_e0)
        pltpu.async_copy(linw_hbm.at[idx_v.at[0]], lin0, sem_l0)

        def pair(i, carry):
            c = 2 * i
            i0 = idx_v.at[c]
            i1 = idx_v.at[c + 1]
            k0, f0 = c // NUM_FIELDS, c % NUM_FIELDS
            k1, f1 = (c + 1) // NUM_FIELDS, (c + 1) % NUM_FIELDS
            pltpu.async_copy(table_hbm.at[i1], rows1, sem_e1)
            pltpu.async_copy(linw_hbm.at[i1], lin1, sem_l1)
            pltpu.make_async_copy(table_hbm.at[i0], rows0, sem_e0).wait()
            pltpu.make_async_copy(linw_hbm.at[i0], lin0, sem_l0).wait()
            pltpu.sync_copy(rows0, emb_out.at[f0, pl.ds(bbase + k0 * CSZ, CSZ)])
            accum(lin0, k0 * CSZ)

            @pl.when(i + 1 < CHUNKS // 2)
            def _():
                i2 = idx_v.at[c + 2]
                pltpu.async_copy(table_hbm.at[i2], rows0, sem_e0)
                pltpu.async_copy(linw_hbm.at[i2], lin0, sem_l0)

            pltpu.make_async_copy(table_hbm.at[i1], rows1, sem_e1).wait()
            pltpu.make_async_copy(linw_hbm.at[i1], lin1, sem_l1).wait()
            pltpu.sync_copy(rows1, emb_out.at[f1, pl.ds(bbase + k1 * CSZ, CSZ)])
            accum(lin1, k1 * CSZ)
            return carry

        lax.fori_loop(0, CHUNKS // 2, pair, 0)
        pltpu.sync_copy(acc_v, lin_out.at[pl.ds(bbase, B_PER_W)])

    return k(w_emb, w_lin_flat, idx3)


def _tc_body(emb_ref, lin_ref, w0_ref, b0_ref, w1_ref, b1_ref, w2_ref,
             c0_ref, ones_ref, out_ref):
    hb = jnp.concatenate(
        [emb_ref[f].astype(jnp.bfloat16) for f in range(NUM_FIELDS)], axis=1)
    y = jnp.dot(hb, w0_ref[...], preferred_element_type=jnp.float32)
    sq = jnp.dot(hb * hb, ones_ref[...],
                 preferred_element_type=jnp.float32)[:, 0:1]
    s = y[:, H0:NAUG]
    fm = 0.5 * (jnp.sum(s * s, axis=1, keepdims=True) - sq)
    y0 = jnp.maximum((y[:, 0:H0] + b0_ref[...]) * BN_SCALE, 0.0)
    y1 = jnp.dot(y0.astype(jnp.bfloat16), w1_ref[...],
                 preferred_element_type=jnp.float32)
    y1 = jnp.maximum((y1 + b1_ref[...]) * BN_SCALE, 0.0)
    y2 = jnp.sum(y1 * w2_ref[...], axis=1, keepdims=True)
    logit = lin_ref[...] + fm + y2 + c0_ref[...]
    out_ref[...] = jax.nn.sigmoid(logit)


def _tc_fused(emb3, lin, w0aug, b0r, w1b, b1r, w2r, c0, block_b,
              interpret=False):
    grid = (BATCH // block_b,)
    ones8 = jnp.ones((EMBED_OUT, 8), dtype=jnp.bfloat16)
    return pl.pallas_call(
        _tc_body,
        grid=grid,
        in_specs=[
            pl.BlockSpec((NUM_FIELDS, block_b, EMBED_DIM), lambda i: (0, i, 0)),
            pl.BlockSpec((block_b, 1), lambda i: (i, 0)),
            pl.BlockSpec((EMBED_OUT, NAUG), lambda i: (0, 0)),
            pl.BlockSpec((1, H0), lambda i: (0, 0)),
            pl.BlockSpec((H0, H1), lambda i: (0, 0)),
            pl.BlockSpec((1, H1), lambda i: (0, 0)),
            pl.BlockSpec((1, H1), lambda i: (0, 0)),
            pl.BlockSpec((1, 1), lambda i: (0, 0)),
            pl.BlockSpec((EMBED_OUT, 8), lambda i: (0, 0)),
        ],
        out_specs=pl.BlockSpec((block_b, 1), lambda i: (i, 0)),
        out_shape=jax.ShapeDtypeStruct((BATCH, 1), jnp.float32),
        interpret=interpret,
    )(emb3, lin, w0aug, b0r, w1b, b1r, w2r, c0, ones8)


def _build_idx3(x):
    xo = x + jnp.asarray(OFFSETS)[None, :]
    return (xo.reshape(NW, KSUB, CSZ, NUM_FIELDS)
            .transpose(0, 1, 3, 2)
            .reshape(NW, CHUNKS, CSZ))


def _build_w0aug(W0):
    smat = jnp.asarray(
        np.tile(np.eye(EMBED_DIM, dtype=np.float32), (NUM_FIELDS, 1)),
        dtype=jnp.bfloat16)
    return jnp.concatenate([W0.astype(jnp.bfloat16), smat], axis=1)


def kernel(x, W_emb, W_lin, bias, W0, b0, W1, b1, W2, b2):
    idx3 = _build_idx3(x)
    emb3, lin = _sc_gather(W_emb, W_lin.reshape(-1), idx3)
    out = _tc_fused(
        emb3, lin.reshape(BATCH, 1),
        _build_w0aug(W0), b0.reshape(1, -1),
        W1.astype(jnp.bfloat16), b1.reshape(1, -1),
        W2.reshape(1, -1), (bias + b2).reshape(1, 1),
        block_b=512,
    )
    return out.reshape(BATCH)


# speedup vs baseline: 2.3492x; 1.1119x over previous
import functools

import jax
import jax.numpy as jnp
import numpy as np
from jax import lax
from jax.experimental import pallas as pl
from jax.experimental.pallas import tpu as pltpu
from jax.experimental.pallas import tpu_sc as plsc

FIELD_DIMS = [100000] * 26
NUM_FIELDS = len(FIELD_DIMS)
TOTAL_DIM = sum(FIELD_DIMS)
EMBED_DIM = 128
BATCH = 16384
EMBED_OUT = NUM_FIELDS * EMBED_DIM
OFFSETS = np.concatenate(([0], np.cumsum(FIELD_DIMS)[:-1])).astype(np.int32)
BN_SCALE = float(1.0 / np.sqrt(1.0 + 1e-5))

NC, NS = 2, 16
NW = NC * NS
CSZ = 128
H0, H1 = 1024, 512
NAUG = H0 + EMBED_DIM


def _sc_gather(w_emb, w_lin_flat, idx3, nb):
    b_per_w = nb // NW
    ksub = b_per_w // CSZ
    chunks = ksub * NUM_FIELDS
    mesh = plsc.VectorSubcoreMesh(core_axis_name="c", subcore_axis_name="s")

    @functools.partial(
        pl.kernel,
        out_type=(
            jax.ShapeDtypeStruct((NUM_FIELDS, nb, EMBED_DIM), jnp.float32),
            jax.ShapeDtypeStruct((nb,), jnp.float32),
        ),
        mesh=mesh,
        scratch_types=(
            pltpu.VMEM((chunks, CSZ), jnp.int32),
            pltpu.VMEM((CSZ, EMBED_DIM), jnp.float32),
            pltpu.VMEM((CSZ, EMBED_DIM), jnp.float32),
            pltpu.VMEM((CSZ,), jnp.float32),
            pltpu.VMEM((CSZ,), jnp.float32),
            pltpu.VMEM((b_per_w,), jnp.float32),
            pltpu.SemaphoreType.DMA,
            pltpu.SemaphoreType.DMA,
            pltpu.SemaphoreType.DMA,
            pltpu.SemaphoreType.DMA,
        ),
    )
    def k(table_hbm, linw_hbm, idx_hbm, emb_out, lin_out,
          idx_v, rows0, rows1, lin0, lin1, acc_v,
          sem_e0, sem_e1, sem_l0, sem_l1):
        wid = lax.axis_index("s") * NC + lax.axis_index("c")
        bbase = wid * b_per_w
        pltpu.sync_copy(idx_hbm.at[wid], idx_v)
        for j in range(b_per_w // 16):
            acc_v[pl.ds(j * 16, 16)] = jnp.zeros((16,), jnp.float32)

        def accum(linbuf, koff):
            for j in range(CSZ // 16):
                sl = pl.ds(koff + j * 16, 16)
                acc_v[sl] = acc_v[sl] + linbuf[pl.ds(j * 16, 16)]

        pltpu.async_copy(table_hbm.at[idx_v.at[0]], rows0, sem_e0)
        pltpu.async_copy(linw_hbm.at[idx_v.at[0]], lin0, sem_l0)

        def pair(i, carry):
            c = 2 * i
            i0 = idx_v.at[c]
            i1 = idx_v.at[c + 1]
            k0, f0 = c // NUM_FIELDS, c % NUM_FIELDS
            k1, f1 = (c + 1) // NUM_FIELDS, (c + 1) % NUM_FIELDS
            pltpu.async_copy(table_hbm.at[i1], rows1, sem_e1)
            pltpu.async_copy(linw_hbm.at[i1], lin1, sem_l1)
            pltpu.make_async_copy(table_hbm.at[i0], rows0, sem_e0).wait()
            pltpu.make_async_copy(linw_hbm.at[i0], lin0, sem_l0).wait()
            pltpu.sync_copy(rows0, emb_out.at[f0, pl.ds(bbase + k0 * CSZ, CSZ)])
            accum(lin0, k0 * CSZ)

            @pl.when(i + 1 < chunks // 2)
            def _():
                i2 = idx_v.at[c + 2]
                pltpu.async_copy(table_hbm.at[i2], rows0, sem_e0)
                pltpu.async_copy(linw_hbm.at[i2], lin0, sem_l0)

            pltpu.make_async_copy(table_hbm.at[i1], rows1, sem_e1).wait()
            pltpu.make_async_copy(linw_hbm.at[i1], lin1, sem_l1).wait()
            pltpu.sync_copy(rows1, emb_out.at[f1, pl.ds(bbase + k1 * CSZ, CSZ)])
            accum(lin1, k1 * CSZ)
            return carry

        lax.fori_loop(0, chunks // 2, pair, 0)
        pltpu.sync_copy(acc_v, lin_out.at[pl.ds(bbase, b_per_w)])

    return k(w_emb, w_lin_flat, idx3)


def _tc_body(emb_ref, lin_ref, w0_ref, b0_ref, w1_ref, b1_ref, w2_ref,
             c0_ref, ones_ref, out_ref):
    hb = jnp.concatenate(
        [emb_ref[f].astype(jnp.bfloat16) for f in range(NUM_FIELDS)], axis=1)
    y = jnp.dot(hb, w0_ref[...], preferred_element_type=jnp.float32)
    sq = jnp.dot(hb * hb, ones_ref[...],
                 preferred_element_type=jnp.float32)[:, 0:1]
    s = y[:, H0:NAUG]
    fm = 0.5 * (jnp.sum(s * s, axis=1, keepdims=True) - sq)
    y0 = jnp.maximum((y[:, 0:H0] + b0_ref[...]) * BN_SCALE, 0.0)
    y1 = jnp.dot(y0.astype(jnp.bfloat16), w1_ref[...],
                 preferred_element_type=jnp.float32)
    y1 = jnp.maximum((y1 + b1_ref[...]) * BN_SCALE, 0.0)
    y2 = jnp.sum(y1 * w2_ref[...], axis=1, keepdims=True)
    logit = lin_ref[...] + fm + y2 + c0_ref[...]
    out_ref[...] = jax.nn.sigmoid(logit)


def _tc_fused(emb3, lin, w0aug, b0r, w1b, b1r, w2r, c0, block_b,
              interpret=False):
    nb = lin.shape[0]
    grid = (nb // block_b,)
    ones8 = jnp.ones((EMBED_OUT, 8), dtype=jnp.bfloat16)
    return pl.pallas_call(
        _tc_body,
        grid=grid,
        in_specs=[
            pl.BlockSpec((NUM_FIELDS, block_b, EMBED_DIM), lambda i: (0, i, 0)),
            pl.BlockSpec((block_b, 1), lambda i: (i, 0)),
            pl.BlockSpec((EMBED_OUT, NAUG), lambda i: (0, 0)),
            pl.BlockSpec((1, H0), lambda i: (0, 0)),
            pl.BlockSpec((H0, H1), lambda i: (0, 0)),
            pl.BlockSpec((1, H1), lambda i: (0, 0)),
            pl.BlockSpec((1, H1), lambda i: (0, 0)),
            pl.BlockSpec((1, 1), lambda i: (0, 0)),
            pl.BlockSpec((EMBED_OUT, 8), lambda i: (0, 0)),
        ],
        out_specs=pl.BlockSpec((block_b, 1), lambda i: (i, 0)),
        out_shape=jax.ShapeDtypeStruct((nb, 1), jnp.float32),
        interpret=interpret,
    )(emb3, lin, w0aug, b0r, w1b, b1r, w2r, c0, ones8)


def _build_idx3(x):
    nb = x.shape[0]
    ksub = nb // NW // CSZ
    xo = x + jnp.asarray(OFFSETS)[None, :]
    return (xo.reshape(NW, ksub, CSZ, NUM_FIELDS)
            .transpose(0, 1, 3, 2)
            .reshape(NW, ksub * NUM_FIELDS, CSZ))


def _build_w0aug(W0):
    smat = jnp.asarray(
        np.tile(np.eye(EMBED_DIM, dtype=np.float32), (NUM_FIELDS, 1)),
        dtype=jnp.bfloat16)
    return jnp.concatenate([W0.astype(jnp.bfloat16), smat], axis=1)


NSPLIT = 2


def kernel(x, W_emb, W_lin, bias, W0, b0, W1, b1, W2, b2):
    w0aug = _build_w0aug(W0)
    b0r = b0.reshape(1, -1)
    w1b = W1.astype(jnp.bfloat16)
    b1r = b1.reshape(1, -1)
    w2r = W2.reshape(1, -1)
    c0 = (bias + b2).reshape(1, 1)
    w_lin_flat = W_lin.reshape(-1)
    nb = BATCH // NSPLIT
    outs = []
    for s in range(NSPLIT):
        xs = x[s * nb:(s + 1) * nb]
        emb3, lin = _sc_gather(W_emb, w_lin_flat, _build_idx3(xs), nb)
        outs.append(_tc_fused(emb3, lin.reshape(nb, 1), w0aug, b0r, w1b,
                              b1r, w2r, c0, block_b=512))
    return jnp.concatenate(outs, axis=0).reshape(BATCH)
